# Initial kernel scaffold; baseline (speedup 1.0000x reference)
import jax
import jax.numpy as jnp
from jax.experimental import pallas as pl


def kernel(coeffs, basis_function_ind, n_basis_per_atom, coeff_ind_to_node_ind):
    raise NotImplementedError("write your pallas kernel here")


# speedup vs baseline: 4.1526x; 4.1526x over previous
import jax
import jax.numpy as jnp
from jax import lax
from jax.experimental import pallas as pl
from jax.experimental.pallas import tpu as pltpu
from jax.experimental.pallas import tpu_sc as plsc

EMBED = 512
N_ATOMS = 50000
N_COEFFS = 6400000

NUM_WORKERS = 32
W_ROWS = 128
W_SLOTS = W_ROWS * EMBED
N_WIN = (N_ATOMS + W_ROWS - 1) // W_ROWS
TAIL_ROWS = N_ATOMS - (N_WIN - 1) * W_ROWS
K_MAX = (N_WIN + NUM_WORKERS - 1) // NUM_WORKERS
CHUNK = 2048
B_PAD = 416


def _sc_body(sk_hbm, sv_hbm, bnd_hbm, out_hbm, buf, kbuf, vbuf, bvec):
    wid = lax.axis_index("s") * 2 + lax.axis_index("c")
    lane = lax.iota(jnp.int32, 16)
    zeros16 = jnp.zeros((16,), jnp.float32)

    pltpu.sync_copy(bnd_hbm, bvec)

    def window_body(k, _):
        w = wid + NUM_WORKERS * k

        @pl.when(w < N_WIN)
        def _():
            w0 = w * W_SLOTS
            bv = bvec[pl.ds(w, 16)]
            s = bv[0]
            e = bv[1]

            def zero_body(i, _):
                buf[pl.ds(i << 4, 16)] = zeros16
                return 0
            lax.fori_loop(0, W_SLOTS // 16, zero_body, 0)

            s8 = s & ~7
            nch = (e - s8 + CHUNK - 1) >> 11

            def chunk_body(c, _):
                st = pl.multiple_of(
                    jnp.minimum(s8 + c * CHUNK, N_COEFFS - CHUNK), 8)
                st2 = pl.multiple_of(
                    jnp.minimum(st + CHUNK, N_COEFFS - 8), 8)
                pltpu.sync_copy(sk_hbm.at[pl.ds(st, CHUNK)],
                                kbuf.at[pl.ds(0, CHUNK)])
                pltpu.sync_copy(sk_hbm.at[pl.ds(st2, 8)],
                                kbuf.at[pl.ds(CHUNK, 8)])
                pltpu.sync_copy(sv_hbm.at[pl.ds(st, CHUNK)], vbuf)

                def vreg_body(j, _):
                    off = j << 4
                    kv = kbuf[pl.ds(off, 16)]
                    kn = kbuf[pl.ds(off + 1, 16)]
                    vv = vbuf[pl.ds(off, 16)]
                    gi = st + off + lane
                    m = (gi >= s) & (gi < e) & (
                        (kv != kn) | (gi == N_COEFFS - 1))
                    idx = kv - w0
                    plsc.store_scatter(buf, [idx], vv, mask=m)
                    return 0
                lax.fori_loop(0, CHUNK // 16, vreg_body, 0)
                return 0
            lax.fori_loop(0, nch, chunk_body, 0)

            @pl.when(w < N_WIN - 1)
            def _():
                pltpu.sync_copy(buf, out_hbm.at[pl.ds(w0, W_SLOTS)])

            @pl.when(w == N_WIN - 1)
            def _():
                pltpu.sync_copy(buf.at[pl.ds(0, TAIL_ROWS * EMBED)],
                                out_hbm.at[pl.ds(w0, TAIL_ROWS * EMBED)])
        return 0

    lax.fori_loop(0, K_MAX, window_body, 0)


def kernel(coeffs, basis_function_ind, n_basis_per_atom, coeff_ind_to_node_ind):
    del n_basis_per_atom
    basis = basis_function_ind.astype(jnp.int32)
    node = coeff_ind_to_node_ind.astype(jnp.int32)

    key = node * EMBED + basis
    sk, sv = lax.sort((key, coeffs), dimension=0, num_keys=1, is_stable=False)

    edges = jnp.arange(B_PAD, dtype=jnp.int32) * W_SLOTS
    bnd = jnp.searchsorted(sk, edges).astype(jnp.int32)

    mesh = plsc.VectorSubcoreMesh(core_axis_name="c", subcore_axis_name="s")
    run = pl.kernel(
        _sc_body,
        out_type=jax.ShapeDtypeStruct((N_ATOMS * EMBED,), jnp.float32),
        mesh=mesh,
        scratch_types=[
            pltpu.VMEM((W_SLOTS,), jnp.float32),
            pltpu.VMEM((CHUNK + 8,), jnp.int32),
            pltpu.VMEM((CHUNK,), jnp.float32),
            pltpu.VMEM((B_PAD,), jnp.int32),
        ],
        compiler_params=pltpu.CompilerParams(needs_layout_passes=False),
    )
    out = run(sk, sv, bnd)
    return out.reshape(N_ATOMS, EMBED)


# speedup vs baseline: 4.4445x; 1.0703x over previous
import jax
import jax.numpy as jnp
from jax import lax
from jax.experimental import pallas as pl
from jax.experimental.pallas import tpu as pltpu
from jax.experimental.pallas import tpu_sc as plsc

EMBED = 512
N_ATOMS = 50000
N_COEFFS = 6400000

NUM_WORKERS = 32
W_ROWS = 128
W_SLOTS = W_ROWS * EMBED
N_WIN = (N_ATOMS + W_ROWS - 1) // W_ROWS
TAIL_ROWS = N_ATOMS - (N_WIN - 1) * W_ROWS
K_MAX = (N_WIN + NUM_WORKERS - 1) // NUM_WORKERS
CHUNK = 4096
CHUNK_SHIFT = 12
B_PAD = 416


def _sc_body(sk_hbm, sv_hbm, bnd_hbm, out_hbm,
             buf, kbuf0, kbuf1, vbuf0, vbuf1, bvec, sem0, sem1):
    wid = lax.axis_index("s") * 2 + lax.axis_index("c")
    lane = lax.iota(jnp.int32, 16)
    zeros16 = jnp.zeros((16,), jnp.float32)
    sems = (sem0, sem1)
    kbufs = (kbuf0, kbuf1)
    vbufs = (vbuf0, vbuf1)

    pltpu.sync_copy(bnd_hbm, bvec)

    def chunk_start(s8, c, b):
        st = pl.multiple_of(jnp.minimum(s8 + c * CHUNK, N_COEFFS - CHUNK), 8)
        st2 = pl.multiple_of(jnp.minimum(st + CHUNK, N_COEFFS - 8), 8)
        pltpu.async_copy(sk_hbm.at[pl.ds(st, CHUNK)],
                         kbufs[b].at[pl.ds(0, CHUNK)], sems[b])
        pltpu.async_copy(sk_hbm.at[pl.ds(st2, 8)],
                         kbufs[b].at[pl.ds(CHUNK, 8)], sems[b])
        pltpu.async_copy(sv_hbm.at[pl.ds(st, CHUNK)], vbufs[b], sems[b])

    def chunk_wait(b):
        pltpu.make_async_copy(sk_hbm.at[pl.ds(0, CHUNK)],
                              kbufs[b].at[pl.ds(0, CHUNK)], sems[b]).wait()
        pltpu.make_async_copy(sk_hbm.at[pl.ds(0, 8)],
                              kbufs[b].at[pl.ds(CHUNK, 8)], sems[b]).wait()
        pltpu.make_async_copy(sv_hbm.at[pl.ds(0, CHUNK)],
                              vbufs[b], sems[b]).wait()

    def window_body(k, _):
        w = wid + NUM_WORKERS * k

        @pl.when(w < N_WIN)
        def _():
            w0 = w * W_SLOTS
            bv = bvec[pl.ds(w, 16)]
            s = bv[0]
            e = bv[1]

            s8 = s & ~7
            nch = (e - s8 + CHUNK - 1) >> CHUNK_SHIFT

            @pl.when(nch > 0)
            def _():
                chunk_start(s8, 0, 0)

            @plsc.parallel_loop(0, W_SLOTS // 16, unroll=8)
            def _(i):
                buf[pl.ds(i << 4, 16)] = zeros16

            def chunk_compute(c, b):
                st = pl.multiple_of(
                    jnp.minimum(s8 + c * CHUNK, N_COEFFS - CHUNK), 8)
                kb = kbufs[b]
                vb = vbufs[b]

                @plsc.parallel_loop(0, CHUNK // 16, unroll=4)
                def _(j):
                    off = j << 4
                    kv = kb[pl.ds(off, 16)]
                    kn = kb[pl.ds(off + 1, 16)]
                    vv = vb[pl.ds(off, 16)]
                    gi = st + off + lane
                    m = (gi >= s) & (gi < e) & (
                        (kv != kn) | (gi == N_COEFFS - 1))
                    idx = kv - w0
                    plsc.store_scatter(buf, [idx], vv, mask=m)

            def pair_body(g, _):
                for b in (0, 1):
                    c = 2 * g + b

                    @pl.when(c < nch)
                    def _():
                        @pl.when(c + 1 < nch)
                        def _():
                            chunk_start(s8, c + 1, (b + 1) % 2)
                        chunk_wait(b)
                        chunk_compute(c, b)
                return 0
            lax.fori_loop(0, (nch + 1) >> 1, pair_body, 0)

            @pl.when(w < N_WIN - 1)
            def _():
                pltpu.sync_copy(buf, out_hbm.at[pl.ds(w0, W_SLOTS)])

            @pl.when(w == N_WIN - 1)
            def _():
                pltpu.sync_copy(buf.at[pl.ds(0, TAIL_ROWS * EMBED)],
                                out_hbm.at[pl.ds(w0, TAIL_ROWS * EMBED)])
        return 0

    lax.fori_loop(0, K_MAX, window_body, 0)


def kernel(coeffs, basis_function_ind, n_basis_per_atom, coeff_ind_to_node_ind):
    del n_basis_per_atom
    basis = basis_function_ind.astype(jnp.int32)
    node = coeff_ind_to_node_ind.astype(jnp.int32)

    key = node * EMBED + basis
    sk, sv = lax.sort((key, coeffs), dimension=0, num_keys=1, is_stable=False)

    edges = jnp.arange(B_PAD, dtype=jnp.int32) * W_SLOTS
    bnd = jnp.searchsorted(sk, edges).astype(jnp.int32)

    mesh = plsc.VectorSubcoreMesh(core_axis_name="c", subcore_axis_name="s")
    run = pl.kernel(
        _sc_body,
        out_type=jax.ShapeDtypeStruct((N_ATOMS * EMBED,), jnp.float32),
        mesh=mesh,
        scratch_types=[
            pltpu.VMEM((W_SLOTS,), jnp.float32),
            pltpu.VMEM((CHUNK + 128,), jnp.int32),
            pltpu.VMEM((CHUNK + 128,), jnp.int32),
            pltpu.VMEM((CHUNK,), jnp.float32),
            pltpu.VMEM((CHUNK,), jnp.float32),
            pltpu.VMEM((B_PAD,), jnp.int32),
            pltpu.SemaphoreType.DMA,
            pltpu.SemaphoreType.DMA,
        ],
        compiler_params=pltpu.CompilerParams(needs_layout_passes=False),
    )
    out = run(sk, sv, bnd)
    return out.reshape(N_ATOMS, EMBED)


# speedup vs baseline: 4.6416x; 1.0443x over previous
import jax
import jax.numpy as jnp
from jax import lax
from jax.experimental import pallas as pl
from jax.experimental.pallas import tpu as pltpu
from jax.experimental.pallas import tpu_sc as plsc

EMBED = 512
N_ATOMS = 50000
N_COEFFS = 6400000

NUM_WORKERS = 32
W_ROWS = 128
W_SLOTS = W_ROWS * EMBED
N_WIN = (N_ATOMS + W_ROWS - 1) // W_ROWS
TAIL_ROWS = N_ATOMS - (N_WIN - 1) * W_ROWS
K_MAX = (N_WIN + NUM_WORKERS - 1) // NUM_WORKERS
CHUNK = 4096
CHUNK_SHIFT = 12
B_PAD = 512

P_TILE = N_COEFFS // NUM_WORKERS
H_NCH = (P_TILE + CHUNK - 1) // CHUNK
I32_MAX = 2147483647


def _bnd_body(key_hbm, bnd_hbm, kb0, kb1, bndv, sem0, sem1):
    wid = lax.axis_index("s") * 2 + lax.axis_index("c")
    lane = lax.iota(jnp.int32, 16)
    sems = (sem0, sem1)
    kbufs = (kb0, kb1)
    t0 = wid * P_TILE
    tend = t0 + P_TILE

    def chunk_start(c, b):
        st = pl.multiple_of(jnp.minimum(t0 + c * CHUNK, N_COEFFS - CHUNK), 8)
        pst = pl.multiple_of(jnp.maximum(st - 8, 0), 8)
        pltpu.async_copy(key_hbm.at[pl.ds(pst, 8)],
                         kbufs[b].at[pl.ds(0, 8)], sems[b])
        pltpu.async_copy(key_hbm.at[pl.ds(st, CHUNK)],
                         kbufs[b].at[pl.ds(8, CHUNK)], sems[b])

    def chunk_wait(b):
        pltpu.make_async_copy(key_hbm.at[pl.ds(0, 8)],
                              kbufs[b].at[pl.ds(0, 8)], sems[b]).wait()
        pltpu.make_async_copy(key_hbm.at[pl.ds(0, CHUNK)],
                              kbufs[b].at[pl.ds(8, CHUNK)], sems[b]).wait()

    maxv = jnp.full((16,), I32_MAX, jnp.int32)

    @plsc.parallel_loop(0, B_PAD // 16, unroll=2)
    def _(i):
        bndv[pl.ds(i << 4, 16)] = maxv

    chunk_start(0, 0)

    def chunk_compute(c, b):
        st = pl.multiple_of(jnp.minimum(t0 + c * CHUNK, N_COEFFS - CHUNK), 8)
        kb = kbufs[b]

        @plsc.parallel_loop(0, CHUNK // 16, unroll=4)
        def _(j):
            off = j << 4
            kp = kb[pl.ds(off + 7, 16)]
            kv = kb[pl.ds(off + 8, 16)]
            gi = st + off + lane
            wv = kv >> 16
            wp = kp >> 16
            m = ((wv > wp) | (gi == 0)) & (gi < tend)
            plsc.store_scatter(bndv, [wv], gi, mask=m)

    def pair_body(g, _):
        for b in (0, 1):
            c = 2 * g + b

            @pl.when(c < H_NCH)
            def _():
                @pl.when(c + 1 < H_NCH)
                def _():
                    chunk_start(c + 1, (b + 1) % 2)
                chunk_wait(b)
                chunk_compute(c, b)
        return 0
    lax.fori_loop(0, (H_NCH + 1) >> 1, pair_body, 0)

    pltpu.sync_copy(bndv, bnd_hbm.at[wid])


def _sc_body(sk_hbm, sv_hbm, bnd_hbm, out_hbm,
             buf, kbuf0, kbuf1, vbuf0, vbuf1, bvec, sem0, sem1):
    wid = lax.axis_index("s") * 2 + lax.axis_index("c")
    lane = lax.iota(jnp.int32, 16)
    zeros16 = jnp.zeros((16,), jnp.float32)
    sems = (sem0, sem1)
    kbufs = (kbuf0, kbuf1)
    vbufs = (vbuf0, vbuf1)

    pltpu.sync_copy(bnd_hbm, bvec)

    def chunk_start(s8, c, b):
        st = pl.multiple_of(jnp.minimum(s8 + c * CHUNK, N_COEFFS - CHUNK), 8)
        st2 = pl.multiple_of(jnp.minimum(st + CHUNK, N_COEFFS - 8), 8)
        pltpu.async_copy(sk_hbm.at[pl.ds(st, CHUNK)],
                         kbufs[b].at[pl.ds(0, CHUNK)], sems[b])
        pltpu.async_copy(sk_hbm.at[pl.ds(st2, 8)],
                         kbufs[b].at[pl.ds(CHUNK, 8)], sems[b])
        pltpu.async_copy(sv_hbm.at[pl.ds(st, CHUNK)], vbufs[b], sems[b])

    def chunk_wait(b):
        pltpu.make_async_copy(sk_hbm.at[pl.ds(0, CHUNK)],
                              kbufs[b].at[pl.ds(0, CHUNK)], sems[b]).wait()
        pltpu.make_async_copy(sk_hbm.at[pl.ds(0, 8)],
                              kbufs[b].at[pl.ds(CHUNK, 8)], sems[b]).wait()
        pltpu.make_async_copy(sv_hbm.at[pl.ds(0, CHUNK)],
                              vbufs[b], sems[b]).wait()

    def window_body(k, _):
        w = wid + NUM_WORKERS * k

        @pl.when(w < N_WIN)
        def _():
            w0 = w * W_SLOTS
            wr = w * W_ROWS
            bv = bvec[pl.ds(w, 16)]
            s = bv[0]
            e = bv[1]

            s8 = s & ~7
            nch = (e - s8 + CHUNK - 1) >> CHUNK_SHIFT

            @pl.when(nch > 0)
            def _():
                chunk_start(s8, 0, 0)

            @plsc.parallel_loop(0, W_SLOTS // 16, unroll=8)
            def _(i):
                buf[i >> 5, pl.ds((i & 31) << 4, 16)] = zeros16

            def chunk_compute(c, b):
                st = pl.multiple_of(
                    jnp.minimum(s8 + c * CHUNK, N_COEFFS - CHUNK), 8)
                kb = kbufs[b]
                vb = vbufs[b]

                @plsc.parallel_loop(0, CHUNK // 16, unroll=4)
                def _(j):
                    off = j << 4
                    kv = kb[pl.ds(off, 16)]
                    kn = kb[pl.ds(off + 1, 16)]
                    vv = vb[pl.ds(off, 16)]
                    gi = st + off + lane
                    m = (gi >= s) & (gi < e) & (
                        (kv != kn) | (gi == N_COEFFS - 1))
                    row = (kv >> 9) - wr
                    col = kv & (EMBED - 1)
                    plsc.store_scatter(buf, [row, col], vv, mask=m)

            def pair_body(g, _):
                for b in (0, 1):
                    c = 2 * g + b

                    @pl.when(c < nch)
                    def _():
                        @pl.when(c + 1 < nch)
                        def _():
                            chunk_start(s8, c + 1, (b + 1) % 2)
                        chunk_wait(b)
                        chunk_compute(c, b)
                return 0
            lax.fori_loop(0, (nch + 1) >> 1, pair_body, 0)

            rr = pl.multiple_of(wr, 8)

            @pl.when(w < N_WIN - 1)
            def _():
                pltpu.sync_copy(buf, out_hbm.at[pl.ds(rr, W_ROWS), :])

            @pl.when(w == N_WIN - 1)
            def _():
                pltpu.sync_copy(buf.at[pl.ds(0, TAIL_ROWS), :],
                                out_hbm.at[pl.ds(rr, TAIL_ROWS), :])
        return 0

    lax.fori_loop(0, K_MAX, window_body, 0)


def kernel(coeffs, basis_function_ind, n_basis_per_atom, coeff_ind_to_node_ind):
    del n_basis_per_atom
    basis = basis_function_ind.astype(jnp.int32)
    node = coeff_ind_to_node_ind.astype(jnp.int32)

    key = node * EMBED + basis

    mesh = plsc.VectorSubcoreMesh(core_axis_name="c", subcore_axis_name="s")

    partials = pl.kernel(
        _bnd_body,
        out_type=jax.ShapeDtypeStruct((NUM_WORKERS, B_PAD), jnp.int32),
        mesh=mesh,
        scratch_types=[
            pltpu.VMEM((CHUNK + 128,), jnp.int32),
            pltpu.VMEM((CHUNK + 128,), jnp.int32),
            pltpu.VMEM((B_PAD,), jnp.int32),
            pltpu.SemaphoreType.DMA,
            pltpu.SemaphoreType.DMA,
        ],
        compiler_params=pltpu.CompilerParams(needs_layout_passes=False),
    )(key)
    col = jnp.minimum(jnp.min(partials, axis=0), N_COEFFS)
    bnd = jnp.flip(jax.lax.cummin(jnp.flip(col)))

    sk, sv = lax.sort((key, coeffs), dimension=0, num_keys=1, is_stable=False)

    out = pl.kernel(
        _sc_body,
        out_type=jax.ShapeDtypeStruct((N_ATOMS, EMBED), jnp.float32),
        mesh=mesh,
        scratch_types=[
            pltpu.VMEM((W_ROWS, EMBED), jnp.float32),
            pltpu.VMEM((CHUNK + 128,), jnp.int32),
            pltpu.VMEM((CHUNK + 128,), jnp.int32),
            pltpu.VMEM((CHUNK,), jnp.float32),
            pltpu.VMEM((CHUNK,), jnp.float32),
            pltpu.VMEM((B_PAD,), jnp.int32),
            pltpu.SemaphoreType.DMA,
            pltpu.SemaphoreType.DMA,
        ],
        compiler_params=pltpu.CompilerParams(needs_layout_passes=False),
    )(sk, sv, bnd)
    return out


# speedup vs baseline: 4.6613x; 1.0042x over previous
import jax
import jax.numpy as jnp
from jax import lax
from jax.experimental import pallas as pl
from jax.experimental.pallas import tpu as pltpu
from jax.experimental.pallas import tpu_sc as plsc

EMBED = 512
N_ATOMS = 50000
N_COEFFS = 6400000

NUM_WORKERS = 32
W_ROWS = 64
W_SHIFT = 15
W_SLOTS = W_ROWS * EMBED
N_WIN = (N_ATOMS + W_ROWS - 1) // W_ROWS
TAIL_ROWS = N_ATOMS - (N_WIN - 1) * W_ROWS
K_MAX = (N_WIN + NUM_WORKERS - 1) // NUM_WORKERS
CHUNK = 4096
CHUNK_SHIFT = 12
B_PAD = 896

P_TILE = N_COEFFS // NUM_WORKERS
H_NCH = (P_TILE + CHUNK - 1) // CHUNK
I32_MAX = 2147483647


def _bnd_body(key_hbm, bnd_hbm, kb0, kb1, bndv, sem0, sem1):
    wid = lax.axis_index("s") * 2 + lax.axis_index("c")
    lane = lax.iota(jnp.int32, 16)
    sems = (sem0, sem1)
    kbufs = (kb0, kb1)
    t0 = wid * P_TILE
    tend = t0 + P_TILE

    def chunk_start(c, b):
        st = pl.multiple_of(jnp.minimum(t0 + c * CHUNK, N_COEFFS - CHUNK), 8)
        pst = pl.multiple_of(jnp.maximum(st - 8, 0), 8)
        pltpu.async_copy(key_hbm.at[pl.ds(pst, 8)],
                         kbufs[b].at[pl.ds(0, 8)], sems[b])
        pltpu.async_copy(key_hbm.at[pl.ds(st, CHUNK)],
                         kbufs[b].at[pl.ds(8, CHUNK)], sems[b])

    def chunk_wait(b):
        pltpu.make_async_copy(key_hbm.at[pl.ds(0, 8)],
                              kbufs[b].at[pl.ds(0, 8)], sems[b]).wait()
        pltpu.make_async_copy(key_hbm.at[pl.ds(0, CHUNK)],
                              kbufs[b].at[pl.ds(8, CHUNK)], sems[b]).wait()

    maxv = jnp.full((16,), I32_MAX, jnp.int32)

    @plsc.parallel_loop(0, B_PAD // 16, unroll=2)
    def _(i):
        bndv[pl.ds(i << 4, 16)] = maxv

    chunk_start(0, 0)

    def chunk_compute(c, b):
        st = pl.multiple_of(jnp.minimum(t0 + c * CHUNK, N_COEFFS - CHUNK), 8)
        kb = kbufs[b]

        @plsc.parallel_loop(0, CHUNK // 16, unroll=4)
        def _(j):
            off = j << 4
            kp = kb[pl.ds(off + 7, 16)]
            kv = kb[pl.ds(off + 8, 16)]
            gi = st + off + lane
            wv = kv >> W_SHIFT
            wp = kp >> W_SHIFT
            m = ((wv > wp) | (gi == 0)) & (gi < tend)
            plsc.store_scatter(bndv, [wv], gi, mask=m)

    def pair_body(g, _):
        for b in (0, 1):
            c = 2 * g + b

            @pl.when(c < H_NCH)
            def _():
                @pl.when(c + 1 < H_NCH)
                def _():
                    chunk_start(c + 1, (b + 1) % 2)
                chunk_wait(b)
                chunk_compute(c, b)
        return 0
    lax.fori_loop(0, (H_NCH + 1) >> 1, pair_body, 0)

    pltpu.sync_copy(bndv, bnd_hbm.at[wid])


def _sc_body(sk_hbm, sv_hbm, bnd_hbm, out_hbm,
             buf0, buf1, kbuf0, kbuf1, vbuf0, vbuf1, bvec,
             sem0, sem1, osem0, osem1):
    wid = lax.axis_index("s") * 2 + lax.axis_index("c")
    lane = lax.iota(jnp.int32, 16)
    zeros16 = jnp.zeros((16,), jnp.float32)
    sems = (sem0, sem1)
    osems = (osem0, osem1)
    kbufs = (kbuf0, kbuf1)
    vbufs = (vbuf0, vbuf1)
    bufs = (buf0, buf1)

    pltpu.sync_copy(bnd_hbm, bvec)

    def chunk_start(s8, c, b):
        st = pl.multiple_of(jnp.minimum(s8 + c * CHUNK, N_COEFFS - CHUNK), 8)
        st2 = pl.multiple_of(jnp.minimum(st + CHUNK, N_COEFFS - 8), 8)
        pltpu.async_copy(sk_hbm.at[pl.ds(st, CHUNK)],
                         kbufs[b].at[pl.ds(0, CHUNK)], sems[b])
        pltpu.async_copy(sk_hbm.at[pl.ds(st2, 8)],
                         kbufs[b].at[pl.ds(CHUNK, 8)], sems[b])
        pltpu.async_copy(sv_hbm.at[pl.ds(st, CHUNK)], vbufs[b], sems[b])

    def chunk_wait(b):
        pltpu.make_async_copy(sk_hbm.at[pl.ds(0, CHUNK)],
                              kbufs[b].at[pl.ds(0, CHUNK)], sems[b]).wait()
        pltpu.make_async_copy(sk_hbm.at[pl.ds(0, 8)],
                              kbufs[b].at[pl.ds(CHUNK, 8)], sems[b]).wait()
        pltpu.make_async_copy(sv_hbm.at[pl.ds(0, CHUNK)],
                              vbufs[b], sems[b]).wait()

    def process_window(k, w, p):
        buf = bufs[p]
        wr = w * W_ROWS
        bv = bvec[pl.ds(w, 16)]
        s = bv[0]
        e = bv[1]

        s8 = s & ~7
        nch = (e - s8 + CHUNK - 1) >> CHUNK_SHIFT

        @pl.when(nch > 0)
        def _():
            chunk_start(s8, 0, 0)

        @pl.when(k >= 2)
        def _():
            pltpu.make_async_copy(
                buf, out_hbm.at[pl.ds(0, W_ROWS), :], osems[p]).wait()

        @plsc.parallel_loop(0, W_SLOTS // 16, unroll=8)
        def _(i):
            buf[i >> 5, pl.ds((i & 31) << 4, 16)] = zeros16

        def chunk_compute(c, b):
            st = pl.multiple_of(
                jnp.minimum(s8 + c * CHUNK, N_COEFFS - CHUNK), 8)
            kb = kbufs[b]
            vb = vbufs[b]

            @plsc.parallel_loop(0, CHUNK // 16, unroll=4)
            def _(j):
                off = j << 4
                kv = kb[pl.ds(off, 16)]
                kn = kb[pl.ds(off + 1, 16)]
                vv = vb[pl.ds(off, 16)]
                gi = st + off + lane
                m = (gi >= s) & (gi < e) & (
                    (kv != kn) | (gi == N_COEFFS - 1))
                row = (kv >> 9) - wr
                col = kv & (EMBED - 1)
                plsc.store_scatter(buf, [row, col], vv, mask=m)

        def pair_body(g, _):
            for b in (0, 1):
                c = 2 * g + b

                @pl.when(c < nch)
                def _():
                    @pl.when(c + 1 < nch)
                    def _():
                        chunk_start(s8, c + 1, (b + 1) % 2)
                    chunk_wait(b)
                    chunk_compute(c, b)
            return 0
        lax.fori_loop(0, (nch + 1) >> 1, pair_body, 0)

        rr = pl.multiple_of(wr, 8)

        @pl.when(w < N_WIN - 1)
        def _():
            pltpu.async_copy(buf, out_hbm.at[pl.ds(rr, W_ROWS), :], osems[p])

        @pl.when(w == N_WIN - 1)
        def _():
            pltpu.sync_copy(buf.at[pl.ds(0, TAIL_ROWS), :],
                            out_hbm.at[pl.ds(rr, TAIL_ROWS), :])

    def wpair_body(g, _):
        for p in (0, 1):
            k = 2 * g + p
            w = wid + NUM_WORKERS * k

            @pl.when(w < N_WIN)
            def _():
                process_window(k, w, p)
        return 0
    lax.fori_loop(0, (K_MAX + 1) >> 1, wpair_body, 0)

    nk = (N_WIN - wid + NUM_WORKERS - 1) // NUM_WORKERS
    for p in (0, 1):
        last_k = nk - 1 - ((nk - 1 - p) % 2)
        last_w = wid + NUM_WORKERS * last_k

        @pl.when((nk > p) & (last_w < N_WIN - 1))
        def _():
            pltpu.make_async_copy(
                bufs[p], out_hbm.at[pl.ds(0, W_ROWS), :], osems[p]).wait()


def kernel(coeffs, basis_function_ind, n_basis_per_atom, coeff_ind_to_node_ind):
    del n_basis_per_atom
    basis = basis_function_ind.astype(jnp.int32)
    node = coeff_ind_to_node_ind.astype(jnp.int32)

    key = node * EMBED + basis

    mesh = plsc.VectorSubcoreMesh(core_axis_name="c", subcore_axis_name="s")

    partials = pl.kernel(
        _bnd_body,
        out_type=jax.ShapeDtypeStruct((NUM_WORKERS, B_PAD), jnp.int32),
        mesh=mesh,
        scratch_types=[
            pltpu.VMEM((CHUNK + 128,), jnp.int32),
            pltpu.VMEM((CHUNK + 128,), jnp.int32),
            pltpu.VMEM((B_PAD,), jnp.int32),
            pltpu.SemaphoreType.DMA,
            pltpu.SemaphoreType.DMA,
        ],
        compiler_params=pltpu.CompilerParams(needs_layout_passes=False),
    )(key)
    col = jnp.minimum(jnp.min(partials, axis=0), N_COEFFS)
    bnd = jnp.flip(jax.lax.cummin(jnp.flip(col)))

    sk, sv = lax.sort((key, coeffs), dimension=0, num_keys=1, is_stable=False)

    out = pl.kernel(
        _sc_body,
        out_type=jax.ShapeDtypeStruct((N_ATOMS, EMBED), jnp.float32),
        mesh=mesh,
        scratch_types=[
            pltpu.VMEM((W_ROWS, EMBED), jnp.float32),
            pltpu.VMEM((W_ROWS, EMBED), jnp.float32),
            pltpu.VMEM((CHUNK + 128,), jnp.int32),
            pltpu.VMEM((CHUNK + 128,), jnp.int32),
            pltpu.VMEM((CHUNK,), jnp.float32),
            pltpu.VMEM((CHUNK,), jnp.float32),
            pltpu.VMEM((B_PAD,), jnp.int32),
            pltpu.SemaphoreType.DMA,
            pltpu.SemaphoreType.DMA,
            pltpu.SemaphoreType.DMA,
            pltpu.SemaphoreType.DMA,
        ],
        compiler_params=pltpu.CompilerParams(needs_layout_passes=False),
    )(sk, sv, bnd)
    return out
